# Initial kernel scaffold; baseline (speedup 1.0000x reference)
import jax
import jax.numpy as jnp
from jax.experimental import pallas as pl


def kernel(inputs_embeds, table):
    raise NotImplementedError("write your pallas kernel here")


# speedup vs baseline: 1.0331x; 1.0331x over previous
import functools

import jax
import jax.numpy as jnp
from jax import lax
from jax.experimental import pallas as pl
from jax.experimental.pallas import tpu as pltpu
from jax.experimental.pallas import tpu_sc as plsc

_POS_OFFSET = 2


@functools.lru_cache(maxsize=None)
def _make_sc_lookup(B, S, D, dtype):
    info = plsc.get_sparse_core_info()
    num_workers = info.num_cores * info.num_subcores
    rows_per_w = S // num_workers
    rb = min(rows_per_w, (64 * 1024) // (D * 4) * 4 or 1)
    rb = max(1, min(rb, 64))
    while rows_per_w % rb:
        rb -= 1
    n_chunks = rows_per_w // rb
    mesh = plsc.VectorSubcoreMesh(core_axis_name="c", subcore_axis_name="s")

    def body(table_hbm, out_hbm, buf):
        wid = lax.axis_index("s") * info.num_cores + lax.axis_index("c")
        base = wid * rows_per_w
        for j in range(n_chunks):
            r0 = base + j * rb
            pltpu.sync_copy(table_hbm.at[pl.ds(r0 + _POS_OFFSET, rb), :], buf)
            for b in range(B):
                pltpu.sync_copy(buf, out_hbm.at[b, pl.ds(r0, rb), :])

    return pl.kernel(
        body,
        out_type=jax.ShapeDtypeStruct((B, S, D), dtype),
        mesh=mesh,
        scratch_types=[pltpu.VMEM((rb, D), dtype)],
        compiler_params=pltpu.CompilerParams(use_tc_tiling_on_sc=False),
    )


@jax.jit
def kernel(inputs_embeds, table):
    B, S, _ = inputs_embeds.shape
    D = table.shape[1]
    return _make_sc_lookup(B, S, D, table.dtype)(table)
